# Initial kernel scaffold; baseline (speedup 1.0000x reference)
import jax
import jax.numpy as jnp
from jax.experimental import pallas as pl


def kernel(x):
    raise NotImplementedError("write your pallas kernel here")


# speedup vs baseline: 1.2087x; 1.2087x over previous
import functools

import jax
import jax.numpy as jnp
from jax import lax
from jax.experimental import pallas as pl
from jax.experimental.pallas import tpu as pltpu
from jax.experimental.pallas import tpu_sc as plsc

BATCH = 16384
NLAB = 256
NGROUP = 16
GSIZE = 16
PERM = [int(s) for s in sorted(str(i) for i in range(NGROUP))]

NC = 2
NS = 16
NW = NC * NS
ROWS_PER_W = BATCH // NW
CHUNK = 128
NCHUNK = ROWS_PER_W // CHUNK
BLOCKS_PER_CHUNK = CHUNK // 16


def _sc_aggregate(xf):
  mesh = plsc.VectorSubcoreMesh(core_axis_name="c", subcore_axis_name="s")

  @functools.partial(
      pl.kernel,
      out_type=jax.ShapeDtypeStruct((BATCH * NGROUP,), jnp.float32),
      mesh=mesh,
      compiler_params=pltpu.CompilerParams(needs_layout_passes=False),
      scratch_types=[
          pltpu.VMEM((CHUNK * NLAB,), jnp.float32),
          pltpu.VMEM((ROWS_PER_W * NGROUP,), jnp.float32),
      ],
  )
  def k(x_hbm, out_hbm, slab, out_slab):
    wid = lax.axis_index("s") * NC + lax.axis_index("c")
    lanes = jnp.arange(16, dtype=jnp.int32)
    row_off = lanes * NLAB
    out_off = lanes * NGROUP

    for ch in range(NCHUNK):
      start = (wid * ROWS_PER_W + ch * CHUNK) * NLAB
      pltpu.sync_copy(x_hbm.at[pl.ds(start, CHUNK * NLAB)], slab)

      def block_body(b, _, ch=ch):
        blk = row_off + b * (16 * NLAB)
        out_blk = out_off + (ch * CHUNK + b * 16) * NGROUP
        for j in range(NGROUP):
          g = PERM[j]
          acc = plsc.load_gather(slab, [blk + (g * GSIZE)])
          for kk in range(1, GSIZE):
            acc = acc + plsc.load_gather(slab, [blk + (g * GSIZE + kk)])
          plsc.store_scatter(out_slab, [out_blk + j], acc)
        return _

      lax.fori_loop(0, BLOCKS_PER_CHUNK, block_body, None)

    pltpu.sync_copy(
        out_slab, out_hbm.at[pl.ds(wid * ROWS_PER_W * NGROUP,
                                   ROWS_PER_W * NGROUP)])

  return k(xf)


@jax.jit
def kernel(x):
  out = _sc_aggregate(x.reshape(-1))
  return out.reshape(BATCH, NGROUP)
